# baseline (speedup 1.0000x reference)
import functools

import jax
import jax.numpy as jnp
from jax import lax
from jax.experimental import pallas as pl
from jax.experimental.pallas import tpu as pltpu
from jax.experimental.pallas import tpu_sc as plsc

N = 10000
D = 128
E = 320000
NC = 2
NS = 16
NW = NC * NS
CHUNK = 128
KB = 8
EPW = -(-E // (NW * CHUNK * KB)) * CHUNK * KB
EPAD = EPW * NW
NCHUNKS = EPW // CHUNK
NBLK = NCHUNKS // KB
TBLK = EPAD // (CHUNK * KB)
BLK0 = 20
BLK1 = (TBLK - NS * BLK0) // NS
NPAD = ((N + NS * CHUNK - 1) // (NS * CHUNK)) * NS * CHUNK
RPT = NPAD // NS
RCH = RPT // CHUNK


def _fill(ref, val):
    v = jnp.full((16,), val, jnp.float32)

    def row(r, _):
        for j in range(D // 16):
            ref[r, pl.ds(j * 16, 16)] = v
        return 0

    lax.fori_loop(0, CHUNK, row, 0, unroll=False)


def _sc_counts_body(dst_hbm3, c0_hbm, c1_hbm, c2_hbm,
                    blkA, blkB, ones128, zero128, acc, ssem):
    cid = lax.axis_index("c")
    sid = lax.axis_index("s")
    _fill(ones128, 1.0)
    _fill(zero128, 0.0)
    r0 = sid * RPT
    wid = cid * NS + sid
    for l, c_hbm in enumerate((c0_hbm, c1_hbm, c2_hbm)):
        def zero(i, _):
            pltpu.sync_copy(zero128, acc.at[pl.ds(r0 + i * CHUNK, CHUNK)])
            return 0
        lax.fori_loop(0, RCH, zero, 0, unroll=False)
        plsc.subcore_barrier()

        def block2(t, _):
            for blk, boff in ((blkA, 0), (blkB, 1)):
                b = wid * NBLK + t * 2 + boff
                pltpu.sync_copy(dst_hbm3.at[l, b], blk)
                copies = [
                    pltpu.async_copy(ones128, acc.at[blk.at[j]], ssem, add=True)
                    for j in range(KB)
                ]
                for cpy in copies:
                    cpy.wait()
            return 0
        lax.fori_loop(0, NBLK // 2, block2, 0, unroll=False)
        plsc.subcore_barrier()

        def out(i, _):
            r = r0 + i * CHUNK
            pltpu.sync_copy(acc.at[pl.ds(r, CHUNK)], c_hbm.at[cid, pl.ds(r, CHUNK)])
            return 0
        lax.fori_loop(0, RCH, out, 0, unroll=False)
        plsc.subcore_barrier()


def _sc_agg_body(x0_hbm, x1_hbm, src_hbm3, dst_hbm3, sums_hbm,
                 sblk, dblk, rows0, rows1, acc, gsem0, gsem1):
    cid = lax.axis_index("c")
    sid = lax.axis_index("s")
    r0 = sid * RPT
    _fill(rows0, 0.0)

    def zero(i, _):
        pltpu.sync_copy(rows0, acc.at[pl.ds(r0 + i * CHUNK, CHUNK)])
        return 0

    lax.fori_loop(0, RCH, zero, 0, unroll=False)
    plsc.subcore_barrier()

    b0 = lax.select(cid == 0, sid * BLK0, NS * BLK0 + sid * BLK1)
    nblk = lax.select(cid == 0, BLK0, BLK1)
    bufs = (rows0, rows1)
    sems = (gsem0, gsem1)

    def make_block(x_hbm):
        def block(b, _):
            pltpu.sync_copy(src_hbm3.at[b0 + b], sblk)
            pltpu.sync_copy(dst_hbm3.at[b0 + b], dblk)
            gathers = [None, None]
            gathers[0] = pltpu.async_copy(x_hbm.at[sblk.at[0]], rows0, gsem0)
            for j in range(KB):
                if j + 1 < KB:
                    gathers[(j + 1) % 2] = pltpu.async_copy(
                        x_hbm.at[sblk.at[j + 1]], bufs[(j + 1) % 2],
                        sems[(j + 1) % 2])
                gathers[j % 2].wait()
                pltpu.sync_copy(bufs[j % 2], acc.at[dblk.at[j]], add=True)
            return 0
        return block

    @pl.when(cid == 0)
    def _():
        lax.fori_loop(0, nblk, make_block(x0_hbm), 0, unroll=False)

    @pl.when(cid == 1)
    def _():
        lax.fori_loop(0, nblk, make_block(x1_hbm), 0, unroll=False)

    plsc.subcore_barrier()

    def out(i, _):
        r = r0 + i * CHUNK
        pltpu.sync_copy(acc.at[pl.ds(r, CHUNK)], sums_hbm.at[cid, pl.ds(r, CHUNK)])
        return 0

    lax.fori_loop(0, RCH, out, 0, unroll=False)


_SC_MESH = plsc.VectorSubcoreMesh(core_axis_name="c", subcore_axis_name="s",
                                  num_cores=NC, num_subcores=NS)

_COUNT_OUT = [jax.ShapeDtypeStruct((NC, NPAD, D), jnp.float32)] * 3
_COUNT_SCRATCH = [
    pltpu.VMEM((KB, CHUNK), jnp.int32),
    pltpu.VMEM((KB, CHUNK), jnp.int32),
    pltpu.VMEM((CHUNK, D), jnp.float32),
    pltpu.VMEM((CHUNK, D), jnp.float32),
    pltpu.VMEM_SHARED((NPAD, D), jnp.float32),
    pltpu.SemaphoreType.DMA,
]

_sc_counts = pl.kernel(
    _sc_counts_body,
    out_type=_COUNT_OUT,
    mesh=_SC_MESH,
    scratch_types=_COUNT_SCRATCH,
)

_AGG_OUT = jax.ShapeDtypeStruct((NC, NPAD, D), jnp.float32)
_AGG_SCRATCH = [
    pltpu.VMEM((KB, CHUNK), jnp.int32),
    pltpu.VMEM((KB, CHUNK), jnp.int32),
    pltpu.VMEM((CHUNK, D), jnp.float32),
    pltpu.VMEM((CHUNK, D), jnp.float32),
    pltpu.VMEM_SHARED((NPAD, D), jnp.float32),
    pltpu.SemaphoreType.DMA,
    pltpu.SemaphoreType.DMA,
]

_sc_aggregate = pl.kernel(
    _sc_agg_body,
    out_type=_AGG_OUT,
    mesh=_SC_MESH,
    scratch_types=_AGG_SCRATCH,
)


BR = 1000


def _tc_body(sums_ref, cnts_ref, x_ref, wl_ref, wr_ref, b_ref,
             o0_ref, o1_ref, *, last):
    s = sums_ref[0] + sums_ref[1]
    c = cnts_ref[0, :, 0] + cnts_ref[1, :, 0]
    mean = s / jnp.maximum(c, 1.0)[:, None]
    out = jnp.dot(mean, wl_ref[...], preferred_element_type=jnp.float32)
    out = out + jnp.dot(x_ref[...], wr_ref[...], preferred_element_type=jnp.float32)
    out = out + b_ref[...]
    if not last:
        out = jax.nn.gelu(out)
    o0_ref[...] = out
    o1_ref[...] = out


def _tc_combine(sums, cnts, x, wl, wr, b, last):
    return pl.pallas_call(
        functools.partial(_tc_body, last=last),
        grid=(N // BR,),
        in_specs=[
            pl.BlockSpec((NC, BR, D), lambda i: (0, i, 0)),
            pl.BlockSpec((NC, BR, D), lambda i: (0, i, 0)),
            pl.BlockSpec((BR, D), lambda i: (i, 0)),
            pl.BlockSpec((D, D), lambda i: (0, 0)),
            pl.BlockSpec((D, D), lambda i: (0, 0)),
            pl.BlockSpec((1, D), lambda i: (0, 0)),
        ],
        out_specs=[pl.BlockSpec((BR, D), lambda i: (i, 0)),
                   pl.BlockSpec((BR, D), lambda i: (i, 0))],
        out_shape=[jax.ShapeDtypeStruct((N, D), jnp.float32),
                   jax.ShapeDtypeStruct((N, D), jnp.float32)],
    )(sums, cnts, x, wl, wr, b)


def _tc_dup_body(x_ref, o0_ref, o1_ref):
    o0_ref[...] = x_ref[...]
    o1_ref[...] = x_ref[...]


def _tc_dup(x):
    return pl.pallas_call(
        _tc_dup_body,
        grid=(N // BR,),
        in_specs=[pl.BlockSpec((BR, D), lambda i: (i, 0))],
        out_specs=[pl.BlockSpec((BR, D), lambda i: (i, 0)),
                   pl.BlockSpec((BR, D), lambda i: (i, 0))],
        out_shape=[jax.ShapeDtypeStruct((N, D), jnp.float32),
                   jax.ShapeDtypeStruct((N, D), jnp.float32)],
    )(x)


def kernel(x, edge_index0, edge_index1, edge_index2,
           Wl0, Wr0, b0, Wl1, Wr1, b1, Wl2, Wr2, b2):
    eis = (edge_index0, edge_index1, edge_index2)
    params = ((Wl0, Wr0, b0), (Wl1, Wr1, b1), (Wl2, Wr2, b2))
    pad = EPAD - E
    srcs = [jnp.concatenate([ei[0], jnp.zeros((pad,), jnp.int32)])
            .reshape(TBLK, KB, CHUNK) for ei in eis]
    dsts = [jnp.concatenate([ei[1], jnp.full((pad,), N, jnp.int32)])
            .reshape(TBLK, KB, CHUNK) for ei in eis]
    cnts = _sc_counts(jnp.stack(dsts))
    x0, x1 = _tc_dup(x)
    for i in range(3):
        sums = _sc_aggregate(x0, x1, srcs[i], dsts[i])
        wl, wr, b = params[i]
        x0, x1 = _tc_combine(sums, cnts[i], x0, wl, wr, b.reshape(1, D),
                             last=(i == 2))
    return x0


# speedup vs baseline: 1.2232x; 1.2232x over previous
import functools

import jax
import jax.numpy as jnp
from jax import lax
from jax.experimental import pallas as pl
from jax.experimental.pallas import tpu as pltpu
from jax.experimental.pallas import tpu_sc as plsc

N = 10000
D = 128
E = 320000
NC = 2
NS = 16
NW = NC * NS
CHUNK = 64
KB = 8
Q = 4
EPW = -(-E // (NW * CHUNK * KB)) * CHUNK * KB
EPAD = EPW * NW
NCHUNKS = EPW // CHUNK
NBLK = NCHUNKS // KB
TBLK = EPAD // (CHUNK * KB)
BLK0 = 31
BLK1 = (TBLK - NS * BLK0) // NS
NPAD = ((N + NS * CHUNK - 1) // (NS * CHUNK)) * NS * CHUNK
RPT = NPAD // NS
RCH = RPT // CHUNK


def _fill(ref, val):
    v = jnp.full((16,), val, jnp.float32)

    def row(r, _):
        for j in range(D // 16):
            ref[r, pl.ds(j * 16, 16)] = v
        return 0

    lax.fori_loop(0, CHUNK, row, 0, unroll=False)


def _sc_counts_body(dst_hbm3, c0_hbm, c1_hbm, c2_hbm,
                    blkA, blkB, ones128, zero128, acc, ssem):
    cid = lax.axis_index("c")
    sid = lax.axis_index("s")
    _fill(ones128, 1.0)
    _fill(zero128, 0.0)
    r0 = sid * RPT
    wid = cid * NS + sid
    for l, c_hbm in enumerate((c0_hbm, c1_hbm, c2_hbm)):
        def zero(i, _):
            pltpu.sync_copy(zero128, acc.at[pl.ds(r0 + i * CHUNK, CHUNK)])
            return 0
        lax.fori_loop(0, RCH, zero, 0, unroll=False)
        plsc.subcore_barrier()

        def block2(t, _):
            for blk, boff in ((blkA, 0), (blkB, 1)):
                b = wid * NBLK + t * 2 + boff
                pltpu.sync_copy(dst_hbm3.at[l, b], blk)
                copies = [
                    pltpu.async_copy(ones128, acc.at[blk.at[j]], ssem, add=True)
                    for j in range(KB)
                ]
                for cpy in copies:
                    cpy.wait()
            return 0
        lax.fori_loop(0, NBLK // 2, block2, 0, unroll=False)
        plsc.subcore_barrier()

        def out(i, _):
            r = r0 + i * CHUNK
            pltpu.sync_copy(acc.at[pl.ds(r, CHUNK)], c_hbm.at[cid, pl.ds(r, CHUNK)])
            return 0
        lax.fori_loop(0, RCH, out, 0, unroll=False)
        plsc.subcore_barrier()


def _sc_agg_body(x0_hbm, x1_hbm, src_hbm3, dst_hbm3, sums_hbm,
                 sblk, dblk, rows0, rows1, rows2, rows3, acc,
                 gsem0, gsem1, gsem2, gsem3):
    cid = lax.axis_index("c")
    sid = lax.axis_index("s")
    r0 = sid * RPT
    _fill(rows0, 0.0)

    def zero(i, _):
        pltpu.sync_copy(rows0, acc.at[pl.ds(r0 + i * CHUNK, CHUNK)])
        return 0

    lax.fori_loop(0, RCH, zero, 0, unroll=False)
    plsc.subcore_barrier()

    b0 = lax.select(cid == 0, sid * BLK0, NS * BLK0 + sid * BLK1)
    nblk = lax.select(cid == 0, BLK0, BLK1)
    bufs = (rows0, rows1, rows2, rows3)
    sems = (gsem0, gsem1, gsem2, gsem3)

    def make_block(x_hbm):
        def block(b, _):
            pltpu.sync_copy(src_hbm3.at[b0 + b], sblk)
            pltpu.sync_copy(dst_hbm3.at[b0 + b], dblk)
            gathers = [None] * Q
            for j in range(Q - 1):
                gathers[j] = pltpu.async_copy(
                    x_hbm.at[sblk.at[j]], bufs[j], sems[j])
            for j in range(KB):
                if j + Q - 1 < KB:
                    gathers[(j + Q - 1) % Q] = pltpu.async_copy(
                        x_hbm.at[sblk.at[j + Q - 1]], bufs[(j + Q - 1) % Q],
                        sems[(j + Q - 1) % Q])
                gathers[j % Q].wait()
                pltpu.sync_copy(bufs[j % Q], acc.at[dblk.at[j]], add=True)
            return 0
        return block

    @pl.when(cid == 0)
    def _():
        lax.fori_loop(0, nblk, make_block(x0_hbm), 0, unroll=False)

    @pl.when(cid == 1)
    def _():
        lax.fori_loop(0, nblk, make_block(x1_hbm), 0, unroll=False)

    plsc.subcore_barrier()

    def out(i, _):
        r = r0 + i * CHUNK
        pltpu.sync_copy(acc.at[pl.ds(r, CHUNK)], sums_hbm.at[cid, pl.ds(r, CHUNK)])
        return 0

    lax.fori_loop(0, RCH, out, 0, unroll=False)


_SC_MESH = plsc.VectorSubcoreMesh(core_axis_name="c", subcore_axis_name="s",
                                  num_cores=NC, num_subcores=NS)

_COUNT_OUT = [jax.ShapeDtypeStruct((NC, NPAD, D), jnp.float32)] * 3
_COUNT_SCRATCH = [
    pltpu.VMEM((KB, CHUNK), jnp.int32),
    pltpu.VMEM((KB, CHUNK), jnp.int32),
    pltpu.VMEM((CHUNK, D), jnp.float32),
    pltpu.VMEM((CHUNK, D), jnp.float32),
    pltpu.VMEM_SHARED((NPAD, D), jnp.float32),
    pltpu.SemaphoreType.DMA,
]

_sc_counts = pl.kernel(
    _sc_counts_body,
    out_type=_COUNT_OUT,
    mesh=_SC_MESH,
    scratch_types=_COUNT_SCRATCH,
)

_AGG_OUT = jax.ShapeDtypeStruct((NC, NPAD, D), jnp.float32)
_AGG_SCRATCH = [
    pltpu.VMEM((KB, CHUNK), jnp.int32),
    pltpu.VMEM((KB, CHUNK), jnp.int32),
    pltpu.VMEM((CHUNK, D), jnp.float32),
    pltpu.VMEM((CHUNK, D), jnp.float32),
    pltpu.VMEM((CHUNK, D), jnp.float32),
    pltpu.VMEM((CHUNK, D), jnp.float32),
    pltpu.VMEM_SHARED((NPAD, D), jnp.float32),
    pltpu.SemaphoreType.DMA,
    pltpu.SemaphoreType.DMA,
    pltpu.SemaphoreType.DMA,
    pltpu.SemaphoreType.DMA,
]

_sc_aggregate = pl.kernel(
    _sc_agg_body,
    out_type=_AGG_OUT,
    mesh=_SC_MESH,
    scratch_types=_AGG_SCRATCH,
)


BR = 1000


def _tc_body(sums_ref, cnts_ref, x_ref, wl_ref, wr_ref, b_ref,
             o0_ref, o1_ref, *, last):
    s = sums_ref[0] + sums_ref[1]
    c = cnts_ref[0, :, 0] + cnts_ref[1, :, 0]
    mean = s / jnp.maximum(c, 1.0)[:, None]
    out = jnp.dot(mean, wl_ref[...], preferred_element_type=jnp.float32)
    out = out + jnp.dot(x_ref[...], wr_ref[...], preferred_element_type=jnp.float32)
    out = out + b_ref[...]
    if not last:
        out = jax.nn.gelu(out)
    o0_ref[...] = out
    o1_ref[...] = out


def _tc_combine(sums, cnts, x, wl, wr, b, last):
    return pl.pallas_call(
        functools.partial(_tc_body, last=last),
        grid=(N // BR,),
        in_specs=[
            pl.BlockSpec((NC, BR, D), lambda i: (0, i, 0)),
            pl.BlockSpec((NC, BR, D), lambda i: (0, i, 0)),
            pl.BlockSpec((BR, D), lambda i: (i, 0)),
            pl.BlockSpec((D, D), lambda i: (0, 0)),
            pl.BlockSpec((D, D), lambda i: (0, 0)),
            pl.BlockSpec((1, D), lambda i: (0, 0)),
        ],
        out_specs=[pl.BlockSpec((BR, D), lambda i: (i, 0)),
                   pl.BlockSpec((BR, D), lambda i: (i, 0))],
        out_shape=[jax.ShapeDtypeStruct((N, D), jnp.float32),
                   jax.ShapeDtypeStruct((N, D), jnp.float32)],
    )(sums, cnts, x, wl, wr, b)


def _tc_dup_body(x_ref, o0_ref, o1_ref):
    o0_ref[...] = x_ref[...]
    o1_ref[...] = x_ref[...]


def _tc_dup(x):
    return pl.pallas_call(
        _tc_dup_body,
        grid=(N // BR,),
        in_specs=[pl.BlockSpec((BR, D), lambda i: (i, 0))],
        out_specs=[pl.BlockSpec((BR, D), lambda i: (i, 0)),
                   pl.BlockSpec((BR, D), lambda i: (i, 0))],
        out_shape=[jax.ShapeDtypeStruct((N, D), jnp.float32),
                   jax.ShapeDtypeStruct((N, D), jnp.float32)],
    )(x)


def kernel(x, edge_index0, edge_index1, edge_index2,
           Wl0, Wr0, b0, Wl1, Wr1, b1, Wl2, Wr2, b2):
    eis = (edge_index0, edge_index1, edge_index2)
    params = ((Wl0, Wr0, b0), (Wl1, Wr1, b1), (Wl2, Wr2, b2))
    pad = EPAD - E
    srcs = [jnp.concatenate([ei[0], jnp.zeros((pad,), jnp.int32)])
            .reshape(TBLK, KB, CHUNK) for ei in eis]
    dsts = [jnp.concatenate([ei[1], jnp.full((pad,), N, jnp.int32)])
            .reshape(TBLK, KB, CHUNK) for ei in eis]
    cnts = _sc_counts(jnp.stack(dsts))
    x0, x1 = _tc_dup(x)
    for i in range(3):
        sums = _sc_aggregate(x0, x1, srcs[i], dsts[i])
        wl, wr, b = params[i]
        x0, x1 = _tc_combine(sums, cnts[i], x0, wl, wr, b.reshape(1, D),
                             last=(i == 2))
    return x0


# speedup vs baseline: 1.2637x; 1.0331x over previous
import functools

import jax
import jax.numpy as jnp
from jax import lax
from jax.experimental import pallas as pl
from jax.experimental.pallas import tpu as pltpu
from jax.experimental.pallas import tpu_sc as plsc

N = 10000
D = 128
E = 320000
NC = 2
NS = 16
NW = NC * NS
CHUNK = 128
KB = 8
Q = 2
EPW = -(-E // (NW * CHUNK * KB)) * CHUNK * KB
EPAD = EPW * NW
NCHUNKS = EPW // CHUNK
NBLK = NCHUNKS // KB
TBLK = EPAD // (CHUNK * KB)
BLK0 = 15
BLK1 = (TBLK - NS * BLK0) // NS
NPAD = ((N + NS * CHUNK - 1) // (NS * CHUNK)) * NS * CHUNK
RPT = NPAD // NS
RCH = RPT // CHUNK


def _fill(ref, val):
    v = jnp.full((16,), val, jnp.float32)

    def row(r, _):
        for j in range(D // 16):
            ref[r, pl.ds(j * 16, 16)] = v
        return 0

    lax.fori_loop(0, CHUNK, row, 0, unroll=False)


def _sc_counts_body(dst_hbm3, c0_hbm, c1_hbm, c2_hbm,
                    blkA, blkB, ones128, zero128, acc, ssem):
    cid = lax.axis_index("c")
    sid = lax.axis_index("s")
    _fill(ones128, 1.0)
    _fill(zero128, 0.0)
    r0 = sid * RPT
    wid = cid * NS + sid
    for l, c_hbm in enumerate((c0_hbm, c1_hbm, c2_hbm)):
        def zero(i, _):
            pltpu.sync_copy(zero128, acc.at[pl.ds(r0 + i * CHUNK, CHUNK)])
            return 0
        lax.fori_loop(0, RCH, zero, 0, unroll=False)
        plsc.subcore_barrier()

        def block2(t, _):
            for blk, boff in ((blkA, 0), (blkB, 1)):
                b = wid * NBLK + t * 2 + boff
                pltpu.sync_copy(dst_hbm3.at[l, b], blk)
                copies = [
                    pltpu.async_copy(ones128, acc.at[blk.at[j]], ssem, add=True)
                    for j in range(KB)
                ]
                for cpy in copies:
                    cpy.wait()
            return 0
        lax.fori_loop(0, NBLK // 2, block2, 0, unroll=False)
        plsc.subcore_barrier()

        def out(i, _):
            r = r0 + i * CHUNK
            pltpu.sync_copy(acc.at[pl.ds(r, CHUNK)], c_hbm.at[cid, pl.ds(r, CHUNK)])
            return 0
        lax.fori_loop(0, RCH, out, 0, unroll=False)
        plsc.subcore_barrier()


def _sc_agg_body(x0_hbm, x1_hbm, src_hbm3, dst_hbm3, sums_hbm,
                 sblk, dblk, rows0, rows1, acc, gsem0, gsem1):
    cid = lax.axis_index("c")
    sid = lax.axis_index("s")
    r0 = sid * RPT
    _fill(rows0, 0.0)

    def zero(i, _):
        pltpu.sync_copy(rows0, acc.at[pl.ds(r0 + i * CHUNK, CHUNK)])
        return 0

    lax.fori_loop(0, RCH, zero, 0, unroll=False)
    plsc.subcore_barrier()

    b0 = lax.select(cid == 0, sid * BLK0, NS * BLK0 + sid * BLK1)
    nblk = lax.select(cid == 0, BLK0, BLK1)
    bufs = (rows0, rows1)
    sems = (gsem0, gsem1)

    def make_block(x_hbm):
        def block(b, _):
            pltpu.sync_copy(src_hbm3.at[b0 + b], sblk)
            pltpu.sync_copy(dst_hbm3.at[b0 + b], dblk)
            gathers = [None] * Q
            for j in range(Q - 1):
                gathers[j] = pltpu.async_copy(
                    x_hbm.at[sblk.at[j]], bufs[j], sems[j])
            for j in range(KB):
                if j + Q - 1 < KB:
                    gathers[(j + Q - 1) % Q] = pltpu.async_copy(
                        x_hbm.at[sblk.at[j + Q - 1]], bufs[(j + Q - 1) % Q],
                        sems[(j + Q - 1) % Q])
                gathers[j % Q].wait()
                pltpu.sync_copy(bufs[j % Q], acc.at[dblk.at[j]], add=True)
            return 0
        return block

    @pl.when(cid == 0)
    def _():
        lax.fori_loop(0, nblk, make_block(x0_hbm), 0, unroll=False)

    @pl.when(cid == 1)
    def _():
        lax.fori_loop(0, nblk, make_block(x1_hbm), 0, unroll=False)

    plsc.subcore_barrier()

    def out(i, _):
        r = r0 + i * CHUNK
        pltpu.sync_copy(acc.at[pl.ds(r, CHUNK)], sums_hbm.at[cid, pl.ds(r, CHUNK)])
        return 0

    lax.fori_loop(0, RCH, out, 0, unroll=False)


_SC_MESH = plsc.VectorSubcoreMesh(core_axis_name="c", subcore_axis_name="s",
                                  num_cores=NC, num_subcores=NS)

_COUNT_OUT = [jax.ShapeDtypeStruct((NC, NPAD, D), jnp.float32)] * 3
_COUNT_SCRATCH = [
    pltpu.VMEM((KB, CHUNK), jnp.int32),
    pltpu.VMEM((KB, CHUNK), jnp.int32),
    pltpu.VMEM((CHUNK, D), jnp.float32),
    pltpu.VMEM((CHUNK, D), jnp.float32),
    pltpu.VMEM_SHARED((NPAD, D), jnp.float32),
    pltpu.SemaphoreType.DMA,
]

_sc_counts = pl.kernel(
    _sc_counts_body,
    out_type=_COUNT_OUT,
    mesh=_SC_MESH,
    scratch_types=_COUNT_SCRATCH,
)

_AGG_OUT = jax.ShapeDtypeStruct((NC, NPAD, D), jnp.float32)
_AGG_SCRATCH = [
    pltpu.VMEM((KB, CHUNK), jnp.int32),
    pltpu.VMEM((KB, CHUNK), jnp.int32),
    pltpu.VMEM((CHUNK, D), jnp.float32),
    pltpu.VMEM((CHUNK, D), jnp.float32),
    pltpu.VMEM_SHARED((NPAD, D), jnp.float32),
    pltpu.SemaphoreType.DMA,
    pltpu.SemaphoreType.DMA,
]

_sc_aggregate = pl.kernel(
    _sc_agg_body,
    out_type=_AGG_OUT,
    mesh=_SC_MESH,
    scratch_types=_AGG_SCRATCH,
)


BR = 1000


def _tc_body(sums_ref, cnts_ref, x_ref, wl_ref, wr_ref, b_ref,
             o0_ref, o1_ref, *, last):
    s = sums_ref[0] + sums_ref[1]
    c = cnts_ref[0, :, 0] + cnts_ref[1, :, 0]
    mean = s / jnp.maximum(c, 1.0)[:, None]
    out = jnp.dot(mean, wl_ref[...], preferred_element_type=jnp.float32)
    out = out + jnp.dot(x_ref[...], wr_ref[...], preferred_element_type=jnp.float32)
    out = out + b_ref[...]
    if not last:
        out = jax.nn.gelu(out)
    o0_ref[...] = out
    o1_ref[...] = out


def _tc_combine(sums, cnts, x, wl, wr, b, last):
    return pl.pallas_call(
        functools.partial(_tc_body, last=last),
        grid=(N // BR,),
        in_specs=[
            pl.BlockSpec((NC, BR, D), lambda i: (0, i, 0)),
            pl.BlockSpec((NC, BR, D), lambda i: (0, i, 0)),
            pl.BlockSpec((BR, D), lambda i: (i, 0)),
            pl.BlockSpec((D, D), lambda i: (0, 0)),
            pl.BlockSpec((D, D), lambda i: (0, 0)),
            pl.BlockSpec((1, D), lambda i: (0, 0)),
        ],
        out_specs=[pl.BlockSpec((BR, D), lambda i: (i, 0)),
                   pl.BlockSpec((BR, D), lambda i: (i, 0))],
        out_shape=[jax.ShapeDtypeStruct((N, D), jnp.float32),
                   jax.ShapeDtypeStruct((N, D), jnp.float32)],
    )(sums, cnts, x, wl, wr, b)


def _tc_dup_body(x_ref, o0_ref, o1_ref):
    o0_ref[...] = x_ref[...]
    o1_ref[...] = x_ref[...]


def _tc_dup(x):
    return pl.pallas_call(
        _tc_dup_body,
        grid=(N // BR,),
        in_specs=[pl.BlockSpec((BR, D), lambda i: (i, 0))],
        out_specs=[pl.BlockSpec((BR, D), lambda i: (i, 0)),
                   pl.BlockSpec((BR, D), lambda i: (i, 0))],
        out_shape=[jax.ShapeDtypeStruct((N, D), jnp.float32),
                   jax.ShapeDtypeStruct((N, D), jnp.float32)],
    )(x)


def kernel(x, edge_index0, edge_index1, edge_index2,
           Wl0, Wr0, b0, Wl1, Wr1, b1, Wl2, Wr2, b2):
    eis = (edge_index0, edge_index1, edge_index2)
    params = ((Wl0, Wr0, b0), (Wl1, Wr1, b1), (Wl2, Wr2, b2))
    pad = EPAD - E
    srcs = [jnp.concatenate([ei[0], jnp.zeros((pad,), jnp.int32)])
            .reshape(TBLK, KB, CHUNK) for ei in eis]
    dsts = [jnp.concatenate([ei[1], jnp.full((pad,), N, jnp.int32)])
            .reshape(TBLK, KB, CHUNK) for ei in eis]
    cnts = _sc_counts(jnp.stack(dsts))
    x0, x1 = _tc_dup(x)
    for i in range(3):
        sums = _sc_aggregate(x0, x1, srcs[i], dsts[i])
        wl, wr, b = params[i]
        x0, x1 = _tc_combine(sums, cnts[i], x0, wl, wr, b.reshape(1, D),
                             last=(i == 2))
    return x0
